# initial kernel scaffold (unmeasured)
import jax
import jax.numpy as jnp
from jax import lax
from jax.experimental import pallas as pl
from jax.experimental.pallas import tpu as pltpu

B = 64
D = 2048
H = 4096
CW = 512
NC_IN = H // CW
NC_OUT = D // CW
NB_IN = 4
NB_OUT = 3
N_LAYERS = 3


def kernel(x, Win0, Wout0, Win1, Wout1, Win2, Wout2):
    def body(x_ref, win0, wout0, win1, wout1, win2, wout2, out_ref,
             winbuf, woutbuf, hbuf, hact, obuf, xact,
             send_y, recv_y, send_x, recv_x,
             win_sems, wout_sems, send_sems, recv_sems):
        my_x = lax.axis_index("x")
        my_y = lax.axis_index("y")
        x_peer = (1 - my_x, my_y)
        y_peer = (my_x, 1 - my_y)

        wins = [win0, win1, win2]
        wouts = [wout0, wout1, wout2]

        def dma_win(l, c):
            return pltpu.make_async_copy(
                wins[l].at[:, pl.ds(c * CW, CW)],
                winbuf.at[c % NB_IN],
                win_sems.at[c % NB_IN],
            )

        def dma_wout(l, c):
            return pltpu.make_async_copy(
                wouts[l].at[:, pl.ds(c * CW, CW)],
                woutbuf.at[c % NB_OUT],
                wout_sems.at[c % NB_OUT],
            )

        barrier = pltpu.get_barrier_semaphore()
        for peer in (x_peer, y_peer):
            pl.semaphore_signal(
                barrier, inc=1, device_id=peer,
                device_id_type=pl.DeviceIdType.MESH,
            )
        pl.semaphore_wait(barrier, 2)

        for c in range(NB_IN - 1):
            dma_win(0, c).start()

        xact[...] = x_ref[...].astype(jnp.bfloat16)

        for l in range(N_LAYERS):
            for c in range(NC_IN):
                if c + NB_IN - 1 < NC_IN:
                    dma_win(l, c + NB_IN - 1).start()
                dma_win(l, c).wait()
                hbuf[:, c * CW:(c + 1) * CW] = jnp.dot(
                    xact[...],
                    winbuf[c % NB_IN].astype(jnp.bfloat16),
                    preferred_element_type=jnp.float32,
                )

            for c in range(NB_OUT - 1):
                dma_wout(l, c).start()

            send_y[...] = hbuf[...].astype(jnp.bfloat16)
            ex = 2 * l
            rdma_y = pltpu.make_async_remote_copy(
                src_ref=send_y,
                dst_ref=recv_y.at[l % 2],
                send_sem=send_sems.at[ex],
                recv_sem=recv_sems.at[ex],
                device_id=y_peer,
                device_id_type=pl.DeviceIdType.MESH,
            )
            rdma_y.start()
            rdma_y.wait()
            hact[...] = jnp.maximum(
                hbuf[...] + recv_y[l % 2].astype(jnp.float32), 0.0
            ).astype(jnp.bfloat16)

            for c in range(NC_OUT):
                if c + NB_OUT - 1 < NC_OUT:
                    dma_wout(l, c + NB_OUT - 1).start()
                dma_wout(l, c).wait()
                obuf[:, c * CW:(c + 1) * CW] = jnp.dot(
                    hact[...],
                    woutbuf[c % NB_OUT].astype(jnp.bfloat16),
                    preferred_element_type=jnp.float32,
                )

            if l + 1 < N_LAYERS:
                for c in range(NB_IN - 1):
                    dma_win(l + 1, c).start()

            send_x[...] = obuf[...].astype(jnp.bfloat16)
            ex = 2 * l + 1
            rdma_x = pltpu.make_async_remote_copy(
                src_ref=send_x,
                dst_ref=recv_x.at[l % 2],
                send_sem=send_sems.at[ex],
                recv_sem=recv_sems.at[ex],
                device_id=x_peer,
                device_id_type=pl.DeviceIdType.MESH,
            )
            rdma_x.start()
            rdma_x.wait()
            if l + 1 < N_LAYERS:
                xact[...] = (
                    obuf[...] + recv_x[l % 2].astype(jnp.float32)
                ).astype(jnp.bfloat16)
            else:
                out_ref[...] = obuf[...] + recv_x[l % 2].astype(jnp.float32)

    return pl.pallas_call(
        body,
        out_shape=jax.ShapeDtypeStruct((B, D), jnp.float32),
        in_specs=[
            pl.BlockSpec(memory_space=pltpu.VMEM),
            pl.BlockSpec(memory_space=pltpu.ANY),
            pl.BlockSpec(memory_space=pltpu.ANY),
            pl.BlockSpec(memory_space=pltpu.ANY),
            pl.BlockSpec(memory_space=pltpu.ANY),
            pl.BlockSpec(memory_space=pltpu.ANY),
            pl.BlockSpec(memory_space=pltpu.ANY),
        ],
        out_specs=pl.BlockSpec(memory_space=pltpu.VMEM),
        scratch_shapes=[
            pltpu.VMEM((NB_IN, D, CW), jnp.float32),
            pltpu.VMEM((NB_OUT, H, CW), jnp.float32),
            pltpu.VMEM((B, H), jnp.float32),
            pltpu.VMEM((B, H), jnp.bfloat16),
            pltpu.VMEM((B, D), jnp.float32),
            pltpu.VMEM((B, D), jnp.bfloat16),
            pltpu.VMEM((B, H), jnp.bfloat16),
            pltpu.VMEM((2, B, H), jnp.bfloat16),
            pltpu.VMEM((B, D), jnp.bfloat16),
            pltpu.VMEM((2, B, D), jnp.bfloat16),
            pltpu.SemaphoreType.DMA((NB_IN,)),
            pltpu.SemaphoreType.DMA((NB_OUT,)),
            pltpu.SemaphoreType.DMA((2 * N_LAYERS,)),
            pltpu.SemaphoreType.DMA((2 * N_LAYERS,)),
        ],
        compiler_params=pltpu.CompilerParams(collective_id=0),
    )(x, Win0, Wout0, Win1, Wout1, Win2, Wout2)


# baseline (device time: 90256 ns/iter reference)
import jax
import jax.numpy as jnp
from jax import lax
from jax.experimental import pallas as pl
from jax.experimental.pallas import tpu as pltpu

B = 64
D = 2048
H = 4096
CW = 512
NC_IN = H // CW
NC_OUT = D // CW
NB_IN = 4
NB_OUT = 3
N_LAYERS = 3


def kernel(x, Win0, Wout0, Win1, Wout1, Win2, Wout2):
    def body(x_ref, win0, wout0, win1, wout1, win2, wout2, out_ref,
             winbuf, woutbuf, hbuf, hact, obuf, xact,
             send_y, recv_y, send_x, recv_x,
             win_sems, wout_sems, send_sems, recv_sems):
        my_x = lax.axis_index("x")
        my_y = lax.axis_index("y")
        x_peer = (1 - my_x, my_y)
        y_peer = (my_x, 1 - my_y)

        wins = [win0, win1, win2]
        wouts = [wout0, wout1, wout2]

        def dma_win(l, c):
            return pltpu.make_async_copy(
                wins[l].at[:, pl.ds(c * CW, CW)],
                winbuf.at[c % NB_IN],
                win_sems.at[c % NB_IN],
            )

        def dma_wout(l, c):
            return pltpu.make_async_copy(
                wouts[l].at[:, pl.ds(c * CW, CW)],
                woutbuf.at[c % NB_OUT],
                wout_sems.at[c % NB_OUT],
            )

        barrier = pltpu.get_barrier_semaphore()
        for peer in (x_peer, y_peer):
            pl.semaphore_signal(
                barrier, inc=1, device_id=peer,
                device_id_type=pl.DeviceIdType.MESH,
            )
        pl.semaphore_wait(barrier, 2)

        for c in range(NB_IN - 1):
            dma_win(0, c).start()

        xact[...] = x_ref[...].astype(jnp.bfloat16)

        for l in range(N_LAYERS):
            for c in range(NC_IN):
                if c + NB_IN - 1 < NC_IN:
                    dma_win(l, c + NB_IN - 1).start()
                dma_win(l, c).wait()
                hbuf[:, c * CW:(c + 1) * CW] = jnp.dot(
                    xact[...],
                    winbuf[c % NB_IN].astype(jnp.bfloat16),
                    preferred_element_type=jnp.float32,
                )

            for c in range(NB_OUT - 1):
                dma_wout(l, c).start()

            send_y[...] = hbuf[...].astype(jnp.bfloat16)
            ex = 2 * l
            rdma_y = pltpu.make_async_remote_copy(
                src_ref=send_y,
                dst_ref=recv_y.at[l % 2],
                send_sem=send_sems.at[ex],
                recv_sem=recv_sems.at[ex],
                device_id=y_peer,
                device_id_type=pl.DeviceIdType.MESH,
            )
            rdma_y.start()
            rdma_y.wait()
            hact[...] = jnp.maximum(
                hbuf[...] + recv_y[l % 2].astype(jnp.float32), 0.0
            ).astype(jnp.bfloat16)

            for c in range(NC_OUT):
                if c + NB_OUT - 1 < NC_OUT:
                    dma_wout(l, c + NB_OUT - 1).start()
                dma_wout(l, c).wait()
                obuf[:, c * CW:(c + 1) * CW] = jnp.dot(
                    hact[...],
                    woutbuf[c % NB_OUT].astype(jnp.bfloat16),
                    preferred_element_type=jnp.float32,
                )

            if l + 1 < N_LAYERS:
                for c in range(NB_IN - 1):
                    dma_win(l + 1, c).start()

            send_x[...] = obuf[...].astype(jnp.bfloat16)
            ex = 2 * l + 1
            rdma_x = pltpu.make_async_remote_copy(
                src_ref=send_x,
                dst_ref=recv_x.at[l % 2],
                send_sem=send_sems.at[ex],
                recv_sem=recv_sems.at[ex],
                device_id=x_peer,
                device_id_type=pl.DeviceIdType.MESH,
            )
            rdma_x.start()
            rdma_x.wait()
            if l + 1 < N_LAYERS:
                xact[...] = (
                    obuf[...] + recv_x[l % 2].astype(jnp.float32)
                ).astype(jnp.bfloat16)
            else:
                out_ref[...] = obuf[...] + recv_x[l % 2].astype(jnp.float32)

    return pl.pallas_call(
        body,
        out_shape=jax.ShapeDtypeStruct((B, D), jnp.float32),
        in_specs=[
            pl.BlockSpec(memory_space=pltpu.VMEM),
            pl.BlockSpec(memory_space=pl.ANY),
            pl.BlockSpec(memory_space=pl.ANY),
            pl.BlockSpec(memory_space=pl.ANY),
            pl.BlockSpec(memory_space=pl.ANY),
            pl.BlockSpec(memory_space=pl.ANY),
            pl.BlockSpec(memory_space=pl.ANY),
        ],
        out_specs=pl.BlockSpec(memory_space=pltpu.VMEM),
        scratch_shapes=[
            pltpu.VMEM((NB_IN, D, CW), jnp.float32),
            pltpu.VMEM((NB_OUT, H, CW), jnp.float32),
            pltpu.VMEM((B, H), jnp.float32),
            pltpu.VMEM((B, H), jnp.bfloat16),
            pltpu.VMEM((B, D), jnp.float32),
            pltpu.VMEM((B, D), jnp.bfloat16),
            pltpu.VMEM((B, H), jnp.bfloat16),
            pltpu.VMEM((2, B, H), jnp.bfloat16),
            pltpu.VMEM((B, D), jnp.bfloat16),
            pltpu.VMEM((2, B, D), jnp.bfloat16),
            pltpu.SemaphoreType.DMA((NB_IN,)),
            pltpu.SemaphoreType.DMA((NB_OUT,)),
            pltpu.SemaphoreType.DMA((2 * N_LAYERS,)),
            pltpu.SemaphoreType.DMA((2 * N_LAYERS,)),
        ],
        compiler_params=pltpu.CompilerParams(
            collective_id=0,
            vmem_limit_bytes=60 * 1024 * 1024,
        ),
    )(x, Win0, Wout0, Win1, Wout1, Win2, Wout2)


# device time: 88607 ns/iter; 1.0186x vs baseline; 1.0186x over previous
import jax
import jax.numpy as jnp
from jax import lax
from jax.experimental import pallas as pl
from jax.experimental.pallas import tpu as pltpu

B = 64
D = 2048
H = 4096
KC_IN = 512
KC_OUT = 1024
NC_IN = D // KC_IN
NC_OUT = H // KC_OUT
NB_IN = 3
NB_OUT = 3
N_LAYERS = 3


def kernel(x, Win0, Wout0, Win1, Wout1, Win2, Wout2):
    def body(x_ref, win0, wout0, win1, wout1, win2, wout2, out_ref,
             winbuf, woutbuf, hbuf, hact, obuf, xact,
             send_y, recv_y, send_x, recv_x,
             win_sems, wout_sems, send_sems, recv_sems):
        my_x = lax.axis_index("x")
        my_y = lax.axis_index("y")
        x_peer = (1 - my_x, my_y)
        y_peer = (my_x, 1 - my_y)

        wins = [win0, win1, win2]
        wouts = [wout0, wout1, wout2]

        def dma_win(l, c):
            return pltpu.make_async_copy(
                wins[l].at[pl.ds(c * KC_IN, KC_IN), :],
                winbuf.at[c % NB_IN],
                win_sems.at[c % NB_IN],
            )

        def dma_wout(l, c):
            return pltpu.make_async_copy(
                wouts[l].at[pl.ds(c * KC_OUT, KC_OUT), :],
                woutbuf.at[c % NB_OUT],
                wout_sems.at[c % NB_OUT],
            )

        barrier = pltpu.get_barrier_semaphore()
        for peer in (x_peer, y_peer):
            pl.semaphore_signal(
                barrier, inc=1, device_id=peer,
                device_id_type=pl.DeviceIdType.MESH,
            )
        pl.semaphore_wait(barrier, 2)

        for c in range(NB_IN - 1):
            dma_win(0, c).start()

        xact[...] = x_ref[...].astype(jnp.bfloat16)

        for l in range(N_LAYERS):
            for c in range(NC_IN):
                if c + NB_IN - 1 < NC_IN:
                    dma_win(l, c + NB_IN - 1).start()
                dma_win(l, c).wait()
                part = jnp.dot(
                    xact[:, c * KC_IN:(c + 1) * KC_IN],
                    winbuf[c % NB_IN].astype(jnp.bfloat16),
                    preferred_element_type=jnp.float32,
                )
                if c == 0:
                    hbuf[...] = part
                else:
                    hbuf[...] += part

            for c in range(NB_OUT - 1):
                dma_wout(l, c).start()

            send_y[...] = hbuf[...].astype(jnp.bfloat16)
            ex = 2 * l
            rdma_y = pltpu.make_async_remote_copy(
                src_ref=send_y,
                dst_ref=recv_y.at[l % 2],
                send_sem=send_sems.at[ex],
                recv_sem=recv_sems.at[ex],
                device_id=y_peer,
                device_id_type=pl.DeviceIdType.MESH,
            )
            rdma_y.start()
            rdma_y.wait()
            hact[...] = jnp.maximum(
                hbuf[...] + recv_y[l % 2].astype(jnp.float32), 0.0
            ).astype(jnp.bfloat16)

            for c in range(NC_OUT):
                if c + NB_OUT - 1 < NC_OUT:
                    dma_wout(l, c + NB_OUT - 1).start()
                dma_wout(l, c).wait()
                part = jnp.dot(
                    hact[:, c * KC_OUT:(c + 1) * KC_OUT],
                    woutbuf[c % NB_OUT].astype(jnp.bfloat16),
                    preferred_element_type=jnp.float32,
                )
                if c == 0:
                    obuf[...] = part
                else:
                    obuf[...] += part

            if l + 1 < N_LAYERS:
                for c in range(NB_IN - 1):
                    dma_win(l + 1, c).start()

            send_x[...] = obuf[...].astype(jnp.bfloat16)
            ex = 2 * l + 1
            rdma_x = pltpu.make_async_remote_copy(
                src_ref=send_x,
                dst_ref=recv_x.at[l % 2],
                send_sem=send_sems.at[ex],
                recv_sem=recv_sems.at[ex],
                device_id=x_peer,
                device_id_type=pl.DeviceIdType.MESH,
            )
            rdma_x.start()
            rdma_x.wait()
            if l + 1 < N_LAYERS:
                xact[...] = (
                    obuf[...] + recv_x[l % 2].astype(jnp.float32)
                ).astype(jnp.bfloat16)
            else:
                out_ref[...] = obuf[...] + recv_x[l % 2].astype(jnp.float32)

    return pl.pallas_call(
        body,
        out_shape=jax.ShapeDtypeStruct((B, D), jnp.float32),
        in_specs=[
            pl.BlockSpec(memory_space=pltpu.VMEM),
            pl.BlockSpec(memory_space=pl.ANY),
            pl.BlockSpec(memory_space=pl.ANY),
            pl.BlockSpec(memory_space=pl.ANY),
            pl.BlockSpec(memory_space=pl.ANY),
            pl.BlockSpec(memory_space=pl.ANY),
            pl.BlockSpec(memory_space=pl.ANY),
        ],
        out_specs=pl.BlockSpec(memory_space=pltpu.VMEM),
        scratch_shapes=[
            pltpu.VMEM((NB_IN, KC_IN, H), jnp.float32),
            pltpu.VMEM((NB_OUT, KC_OUT, D), jnp.float32),
            pltpu.VMEM((B, H), jnp.float32),
            pltpu.VMEM((B, H), jnp.bfloat16),
            pltpu.VMEM((B, D), jnp.float32),
            pltpu.VMEM((B, D), jnp.bfloat16),
            pltpu.VMEM((B, H), jnp.bfloat16),
            pltpu.VMEM((2, B, H), jnp.bfloat16),
            pltpu.VMEM((B, D), jnp.bfloat16),
            pltpu.VMEM((2, B, D), jnp.bfloat16),
            pltpu.SemaphoreType.DMA((NB_IN,)),
            pltpu.SemaphoreType.DMA((NB_OUT,)),
            pltpu.SemaphoreType.DMA((2 * N_LAYERS,)),
            pltpu.SemaphoreType.DMA((2 * N_LAYERS,)),
        ],
        compiler_params=pltpu.CompilerParams(
            collective_id=0,
            vmem_limit_bytes=62 * 1024 * 1024,
        ),
    )(x, Win0, Wout0, Win1, Wout1, Win2, Wout2)
